# baseline (speedup 1.0000x reference)
import functools

import jax
import jax.numpy as jnp
from jax import lax
from jax.experimental import pallas as pl
from jax.experimental.pallas import tpu as pltpu
from jax.experimental.pallas import tpu_sc as plsc

B = 4096
N = 2048
IN_BITS = 1024
K = 6
TBL = 64

NC = 2
NS = 16
NW = NC * NS

BG = 8
LG = 4
RW = B // BG
LW = N // LG
BC = 16
NCHUNK = RW // BC
NG = LW // 16


def _body(x_hbm, tab_hbm, conn_hbm, out_hbm,
          x_v0, x_v1, out_v0, out_v1, tab_v, conn_raw, conn_v,
          sx0, sx1, so0, so1):
    cid = lax.axis_index("c")
    sid = lax.axis_index("s")
    wid = cid * NS + sid
    bg = wid // LG
    lg = wid % LG
    row0 = bg * RW
    lut0 = lg * LW

    xb = [x_v0, x_v1]
    ob = [out_v0, out_v1]
    sx = [sx0, sx1]
    so = [so0, so1]

    lanes = jnp.arange(16, dtype=jnp.int32)
    half = jnp.full((16,), 0.5, jnp.float32)
    one = jnp.full((16,), 1.0, jnp.float32)
    zero = jnp.zeros((16,), jnp.int32)
    wvec = [jnp.full((16,), 32 >> j, jnp.int32) for j in range(K)]

    pltpu.sync_copy(conn_hbm.at[pl.ds(lut0, LW)], conn_raw)
    for g in range(NG):
        g16 = g * 16
        ids = g16 + lanes
        for j in range(K):
            jv = jnp.full((16,), j, jnp.int32)
            c = plsc.load_gather(conn_raw, [ids, jv])
            conn_v[j, pl.ds(g16, 16)] = (
                ((c >> 7) << 10) + (c & 127))
    pltpu.sync_copy(tab_hbm.at[pl.ds(lut0, LW)], tab_v)

    @plsc.parallel_loop(0, LW * (TBL // 16), unroll=16)
    def _sig_body(i):
        row = i // (TBL // 16)
        part = (i % (TBL // 16)) * 16
        t = tab_v[row, pl.ds(part, 16)]
        tab_v[row, pl.ds(part, 16)] = one / (one + jnp.exp(-t))

    def compute_chunk(x_v, out_v):
        def g_body(g, _):
            g16 = g * 16
            lut_ids = g16 + lanes
            cols = [conn_v[j, pl.ds(g16, 16)] for j in range(K)]

            @plsc.parallel_loop(0, BC, unroll=16)
            def r_body(r):
                bits = []
                base = (r // 8) * (8 * IN_BITS) + (r % 8) * 128
                span = 7 * 1024 + 128
                for j in range(K):
                    vals = plsc.load_gather(
                        x_v.at[pl.ds(base, span)], [cols[j]])
                    bits.append(jnp.where(vals > half, wvec[j], zero))
                a01 = bits[0] + bits[1]
                a23 = bits[2] + bits[3]
                a45 = bits[4] + bits[5]
                addr = (a01 + a23) + a45
                out_v[r, pl.ds(g16, 16)] = plsc.load_gather(
                    tab_v, [lut_ids, addr])

            return 0

        lax.fori_loop(0, NG, g_body, 0)

    rmax = row0 + RW - BC

    def x_copy(r0, buf, sem):
        return pltpu.async_copy(
            x_hbm.at[pl.ds(r0 * IN_BITS, BC * IN_BITS)], buf, sem)

    def out_copy(r0, buf, sem):
        h = None
        for a in range(BC // 8):
            for b in range(LW // 128):
                h = pltpu.async_copy(
                    buf.at[pl.ds(a * 8, 8), pl.ds(b * 128, 128)],
                    out_hbm.at[r0 // 8 + a, lut0 // 128 + b], sem)
        return h

    x_copy(row0, xb[0], sx[0])

    def c2_body(c2, _):
        ra = row0 + (2 * c2) * BC
        rb = ra + BC
        rc = jnp.minimum(rb + BC, rmax)

        pltpu.make_async_copy(
            x_hbm.at[pl.ds(ra * IN_BITS, BC * IN_BITS)], xb[0], sx[0]).wait()
        x_copy(rb, xb[1], sx[1])

        @pl.when(c2 > 0)
        def _():
            for a in range(BC // 8):
                for b in range(LW // 128):
                    pltpu.make_async_copy(
                        ob[0].at[pl.ds(a * 8, 8), pl.ds(b * 128, 128)],
                        out_hbm.at[ra // 8 + a, lut0 // 128 + b],
                        so[0]).wait()

        compute_chunk(xb[0], ob[0])
        out_copy(ra, ob[0], so[0])

        pltpu.make_async_copy(
            x_hbm.at[pl.ds(rb * IN_BITS, BC * IN_BITS)], xb[1], sx[1]).wait()
        x_copy(rc, xb[0], sx[0])

        @pl.when(c2 > 0)
        def _():
            for a in range(BC // 8):
                for b in range(LW // 128):
                    pltpu.make_async_copy(
                        ob[1].at[pl.ds(a * 8, 8), pl.ds(b * 128, 128)],
                        out_hbm.at[rb // 8 + a, lut0 // 128 + b],
                        so[1]).wait()

        compute_chunk(xb[1], ob[1])
        out_copy(rb, ob[1], so[1])
        return 0

    lax.fori_loop(0, NCHUNK // 2, c2_body, 0)

    pltpu.make_async_copy(
        x_hbm.at[pl.ds(rmax * IN_BITS, BC * IN_BITS)], xb[0], sx[0]).wait()
    rlast = row0 + RW - 2 * BC
    for a in range(BC // 8):
        for b in range(LW // 128):
            pltpu.make_async_copy(
                ob[0].at[pl.ds(a * 8, 8), pl.ds(b * 128, 128)],
                out_hbm.at[rlast // 8 + a, lut0 // 128 + b], so[0]).wait()
            pltpu.make_async_copy(
                ob[1].at[pl.ds(a * 8, 8), pl.ds(b * 128, 128)],
                out_hbm.at[(rlast + BC) // 8 + a, lut0 // 128 + b],
                so[1]).wait()


@jax.jit
def kernel(x_bits, table, conn_idx):
    mesh = plsc.VectorSubcoreMesh(core_axis_name="c", subcore_axis_name="s")
    run = functools.partial(
        pl.kernel,
        mesh=mesh,
        compiler_params=pltpu.CompilerParams(use_tc_tiling_on_sc=False,
                                             needs_layout_passes=False),
        out_type=jax.ShapeDtypeStruct((B // 8, N // 128, 8, 128),
                                      jnp.float32),
        scratch_types=[
            pltpu.VMEM((BC * IN_BITS,), jnp.float32),
            pltpu.VMEM((BC * IN_BITS,), jnp.float32),
            pltpu.VMEM((BC, LW), jnp.float32),
            pltpu.VMEM((BC, LW), jnp.float32),
            pltpu.VMEM((LW, TBL), jnp.float32),
            pltpu.VMEM((LW, K), jnp.int32),
            pltpu.VMEM((K, LW), jnp.int32),
            pltpu.SemaphoreType.DMA,
            pltpu.SemaphoreType.DMA,
            pltpu.SemaphoreType.DMA,
            pltpu.SemaphoreType.DMA,
        ],
    )(_body)
    out4 = run(x_bits.reshape(B * IN_BITS), table, conn_idx)
    return out4.transpose(0, 2, 1, 3).reshape(B, N)


# speedup vs baseline: 1.0142x; 1.0142x over previous
import functools

import jax
import jax.numpy as jnp
from jax import lax
from jax.experimental import pallas as pl
from jax.experimental.pallas import tpu as pltpu
from jax.experimental.pallas import tpu_sc as plsc

B = 4096
N = 2048
IN_BITS = 1024
K = 6
TBL = 64

NC = 2
NS = 16
NW = NC * NS

BG = 8
LG = 4
RW = B // BG
LW = N // LG
BC = 16
NCHUNK = RW // BC
NG = LW // 16


def _body(x_hbm, tab_hbm, conn_hbm, out_hbm,
          x_v0, x_v1, out_v0, out_v1, tab_v, conn_raw, conn_v,
          sx0, sx1, so0, so1):
    cid = lax.axis_index("c")
    sid = lax.axis_index("s")
    wid = cid * NS + sid
    bg = wid // LG
    lg = wid % LG
    row0 = bg * RW
    lut0 = lg * LW

    xb = [x_v0, x_v1]
    ob = [out_v0, out_v1]
    sx = [sx0, sx1]
    so = [so0, so1]

    lanes = jnp.arange(16, dtype=jnp.int32)
    half = jnp.full((16,), 0.5, jnp.float32)
    one = jnp.full((16,), 1.0, jnp.float32)
    zero = jnp.zeros((16,), jnp.int32)
    wvec = [jnp.full((16,), 32 >> j, jnp.int32) for j in range(K)]

    pltpu.sync_copy(conn_hbm.at[pl.ds(lut0, LW)], conn_raw)
    for g in range(NG):
        g16 = g * 16
        ids = g16 + lanes
        for j in range(K):
            jv = jnp.full((16,), j, jnp.int32)
            conn_v[j, pl.ds(g16, 16)] = plsc.load_gather(conn_raw, [ids, jv])
    pltpu.sync_copy(tab_hbm.at[pl.ds(lut0, LW)], tab_v)

    @plsc.parallel_loop(0, LW * (TBL // 16), unroll=16)
    def _sig_body(i):
        row = i // (TBL // 16)
        part = (i % (TBL // 16)) * 16
        t = tab_v[row, pl.ds(part, 16)]
        tab_v[row, pl.ds(part, 16)] = one / (one + jnp.exp(-t))

    def compute_chunk(x_v, out_v):
        def g_body(g, _):
            g16 = g * 16
            lut_ids = g16 + lanes
            cols = [conn_v[j, pl.ds(g16, 16)] for j in range(K)]

            @plsc.parallel_loop(0, BC, unroll=16)
            def r_body(r):
                bits = []
                for j in range(K):
                    vals = plsc.load_gather(
                        x_v.at[pl.ds(r * IN_BITS, IN_BITS)], [cols[j]])
                    bits.append(jnp.where(vals > half, wvec[j], zero))
                a01 = bits[0] + bits[1]
                a23 = bits[2] + bits[3]
                a45 = bits[4] + bits[5]
                addr = (a01 + a23) + a45
                out_v[r, pl.ds(g16, 16)] = plsc.load_gather(
                    tab_v, [lut_ids, addr])

            return 0

        lax.fori_loop(0, NG, g_body, 0)

    rmax = row0 + RW - BC

    def x_copy(r0, buf, sem):
        return pltpu.async_copy(
            x_hbm.at[pl.ds(r0 * IN_BITS, BC * IN_BITS)], buf, sem)

    def out_copy(r0, buf, sem):
        h = None
        for a in range(BC // 8):
            for b in range(LW // 128):
                h = pltpu.async_copy(
                    buf.at[pl.ds(a * 8, 8), pl.ds(b * 128, 128)],
                    out_hbm.at[r0 // 8 + a, lut0 // 128 + b], sem)
        return h

    x_copy(row0, xb[0], sx[0])

    def c2_body(c2, _):
        ra = row0 + (2 * c2) * BC
        rb = ra + BC
        rc = jnp.minimum(rb + BC, rmax)

        pltpu.make_async_copy(
            x_hbm.at[pl.ds(ra * IN_BITS, BC * IN_BITS)], xb[0], sx[0]).wait()
        x_copy(rb, xb[1], sx[1])

        @pl.when(c2 > 0)
        def _():
            for a in range(BC // 8):
                for b in range(LW // 128):
                    pltpu.make_async_copy(
                        ob[0].at[pl.ds(a * 8, 8), pl.ds(b * 128, 128)],
                        out_hbm.at[ra // 8 + a, lut0 // 128 + b],
                        so[0]).wait()

        compute_chunk(xb[0], ob[0])
        out_copy(ra, ob[0], so[0])

        pltpu.make_async_copy(
            x_hbm.at[pl.ds(rb * IN_BITS, BC * IN_BITS)], xb[1], sx[1]).wait()
        x_copy(rc, xb[0], sx[0])

        @pl.when(c2 > 0)
        def _():
            for a in range(BC // 8):
                for b in range(LW // 128):
                    pltpu.make_async_copy(
                        ob[1].at[pl.ds(a * 8, 8), pl.ds(b * 128, 128)],
                        out_hbm.at[rb // 8 + a, lut0 // 128 + b],
                        so[1]).wait()

        compute_chunk(xb[1], ob[1])
        out_copy(rb, ob[1], so[1])
        return 0

    lax.fori_loop(0, NCHUNK // 2, c2_body, 0)

    pltpu.make_async_copy(
        x_hbm.at[pl.ds(rmax * IN_BITS, BC * IN_BITS)], xb[0], sx[0]).wait()
    rlast = row0 + RW - 2 * BC
    for a in range(BC // 8):
        for b in range(LW // 128):
            pltpu.make_async_copy(
                ob[0].at[pl.ds(a * 8, 8), pl.ds(b * 128, 128)],
                out_hbm.at[rlast // 8 + a, lut0 // 128 + b], so[0]).wait()
            pltpu.make_async_copy(
                ob[1].at[pl.ds(a * 8, 8), pl.ds(b * 128, 128)],
                out_hbm.at[(rlast + BC) // 8 + a, lut0 // 128 + b],
                so[1]).wait()


@jax.jit
def kernel(x_bits, table, conn_idx):
    mesh = plsc.VectorSubcoreMesh(core_axis_name="c", subcore_axis_name="s")
    run = functools.partial(
        pl.kernel,
        mesh=mesh,
        compiler_params=pltpu.CompilerParams(use_tc_tiling_on_sc=False,
                                             needs_layout_passes=False),
        out_type=jax.ShapeDtypeStruct((B // 8, N // 128, 8, 128),
                                      jnp.float32),
        scratch_types=[
            pltpu.VMEM((BC * IN_BITS,), jnp.float32),
            pltpu.VMEM((BC * IN_BITS,), jnp.float32),
            pltpu.VMEM((BC, LW), jnp.float32),
            pltpu.VMEM((BC, LW), jnp.float32),
            pltpu.VMEM((LW, TBL), jnp.float32),
            pltpu.VMEM((LW, K), jnp.int32),
            pltpu.VMEM((K, LW), jnp.int32),
            pltpu.SemaphoreType.DMA,
            pltpu.SemaphoreType.DMA,
            pltpu.SemaphoreType.DMA,
            pltpu.SemaphoreType.DMA,
        ],
    )(_body)
    out4 = run(x_bits.reshape(B * IN_BITS), table, conn_idx)
    return out4.transpose(0, 2, 1, 3).reshape(B, N)
